# baseline (speedup 1.0000x reference)
import functools

import jax
import jax.numpy as jnp
from jax import lax
from jax.experimental import pallas as pl
from jax.experimental.pallas import tpu as pltpu
from jax.experimental.pallas import tpu_sc as plsc

_NE = 8192
_D = 256
_NT = 8192
_COMMIT = 0.25

_TB = 512
_CB = 2048


def _argmin_body(x_ref, w_ref, mv_ref, mi_ref, mv_s, mi_s):
    j = pl.program_id(1)
    nj = pl.num_programs(1)
    x = x_ref[...]
    w = w_ref[pl.ds(j * _CB, _CB), :]
    x2 = jnp.sum(x * x, axis=1, keepdims=True)
    w2 = jnp.sum(w * w, axis=1)
    mm = lax.dot_general(x, w, (((1,), (1,)), ((), ())),
                         preferred_element_type=jnp.float32)
    dist = (x2 + w2[None, :]) - 2.0 * mm

    tmin = jnp.min(dist, axis=1)
    cols = lax.broadcasted_iota(jnp.int32, dist.shape, 1)
    targ = jnp.min(jnp.where(dist == tmin[:, None], cols, _NE),
                   axis=1) + j * _CB

    @pl.when(j == 0)
    def _():
        mv_s[...] = tmin
        mi_s[...] = targ

    @pl.when(j > 0)
    def _():
        better = tmin < mv_s[...]
        mv_s[...] = jnp.where(better, tmin, mv_s[...])
        mi_s[...] = jnp.where(better, targ, mi_s[...])

    @pl.when(j == nj - 1)
    def _():
        mv_ref[...] = mv_s[...]
        mi_ref[...] = mi_s[...]


def _argmin_call(flat_x, weight):
    return pl.pallas_call(
        _argmin_body,
        grid=(_NT // _TB, _NE // _CB),
        in_specs=[
            pl.BlockSpec((_TB, _D), lambda i, j: (i, 0)),
            pl.BlockSpec((_NE, _D), lambda i, j: (0, 0)),
        ],
        out_specs=[
            pl.BlockSpec((_TB,), lambda i, j: (i,)),
            pl.BlockSpec((_TB,), lambda i, j: (i,)),
        ],
        out_shape=[
            jax.ShapeDtypeStruct((_NT,), jnp.float32),
            jax.ShapeDtypeStruct((_NT,), jnp.int32),
        ],
        scratch_shapes=[
            pltpu.VMEM((_TB,), jnp.float32),
            pltpu.VMEM((_TB,), jnp.int32),
        ],
    )(flat_x, weight)


def _dist_body(x_ref, w_ref, mi_ref, mv_ref, dist_ref, loss_ref, ppl_ref,
               cnt_s, acc_s):
    i = pl.program_id(0)
    j = pl.program_id(1)
    ni = pl.num_programs(0)
    nj = pl.num_programs(1)
    x = x_ref[...]
    w = w_ref[pl.ds(j * _CB, _CB), :]
    x2 = jnp.sum(x * x, axis=1, keepdims=True)
    w2 = jnp.sum(w * w, axis=1)
    mm = lax.dot_general(x, w, (((1,), (1,)), ((), ())),
                         preferred_element_type=jnp.float32)
    dist_ref[...] = (x2 + w2[None, :]) - 2.0 * mm

    idx = mi_ref[...]
    cols = j * _CB + lax.broadcasted_iota(jnp.int32, (_TB, _CB), 1)
    colsum = jnp.sum((idx[:, None] == cols).astype(jnp.float32), axis=0)

    @pl.when(i == 0)
    def _():
        cnt_s[pl.ds(j * _CB, _CB)] = colsum

    @pl.when(i > 0)
    def _():
        cnt_s[pl.ds(j * _CB, _CB)] = cnt_s[pl.ds(j * _CB, _CB)] + colsum

    @pl.when(jnp.logical_and(i == 0, j == 0))
    def _():
        acc_s[0] = 0.0

    @pl.when(j == 0)
    def _():
        acc_s[0] = acc_s[0] + jnp.sum(mv_ref[...])

    @pl.when(jnp.logical_and(i == ni - 1, j == nj - 1))
    def _():
        loss_ref[0, 0] = (1.0 + _COMMIT) * acc_s[0] * (1.0 / (_NT * _D))
        p = cnt_s[...] * (1.0 / _NT)
        ppl_ref[0, 0] = jnp.exp(-jnp.sum(p * jnp.log(p + 1e-10)))


def _dist_stats_call(flat_x, weight, minidx, minval):
    return pl.pallas_call(
        _dist_body,
        grid=(_NT // _TB, _NE // _CB),
        in_specs=[
            pl.BlockSpec((_TB, _D), lambda i, j: (i, 0)),
            pl.BlockSpec((_NE, _D), lambda i, j: (0, 0)),
            pl.BlockSpec((_TB,), lambda i, j: (i,)),
            pl.BlockSpec((_TB,), lambda i, j: (i,)),
        ],
        out_specs=[
            pl.BlockSpec((_TB, _CB), lambda i, j: (i, j)),
            pl.BlockSpec(memory_space=pltpu.SMEM),
            pl.BlockSpec(memory_space=pltpu.SMEM),
        ],
        out_shape=[
            jax.ShapeDtypeStruct((_NT, _NE), jnp.float32),
            jax.ShapeDtypeStruct((1, 1), jnp.float32),
            jax.ShapeDtypeStruct((1, 1), jnp.float32),
        ],
        scratch_shapes=[
            pltpu.VMEM((_NE,), jnp.float32),
            pltpu.SMEM((2,), jnp.float32),
        ],
    )(flat_x, weight, minidx, minval)


def _sc_encode_gather(weight, minidx):
    info = plsc.get_sparse_core_info()
    nw = info.num_cores * info.num_subcores
    bpw = _NT // nw
    nchunks = bpw // 128
    zlen = 32768
    per_w = bpw * _NE
    nz = per_w // zlen
    idx2 = minidx.reshape(_NT // 128, 128)
    mesh = plsc.VectorSubcoreMesh(core_axis_name="c", subcore_axis_name="s")

    @functools.partial(
        pl.kernel, mesh=mesh,
        out_type=[
            jax.ShapeDtypeStruct((_NT * _NE,), jnp.float32),
            jax.ShapeDtypeStruct((_NT, _D), jnp.float32),
        ],
        scratch_types=[
            pltpu.VMEM((zlen,), jnp.float32),
            pltpu.VMEM((nchunks, 128), jnp.int32),
            pltpu.VMEM((nchunks, 128), jnp.int32),
            pltpu.VMEM((128,), jnp.float32),
            pltpu.VMEM((bpw, _D), jnp.float32),
            pltpu.SemaphoreType.DMA,
        ],
    )
    def k(w_hbm, idx_hbm, enc_hbm, quant_hbm,
          zeros_v, idx_v, pos_v, ones_v, rows_v, sem):
        wid = lax.axis_index("s") * info.num_cores + lax.axis_index("c")
        tok0 = wid * bpw

        def fill_zeros(t, _):
            zeros_v[pl.ds(t * 16, 16)] = jnp.zeros((16,), jnp.float32)
            return 0

        lax.fori_loop(0, zlen // 16, fill_zeros, 0)
        for c in range(8):
            ones_v[pl.ds(c * 16, 16)] = jnp.ones((16,), jnp.float32)

        pltpu.sync_copy(idx_hbm.at[pl.ds(wid * nchunks, nchunks)], idx_v)

        for r in range(nchunks):
            pltpu.async_copy(w_hbm.at[idx_v.at[r]],
                             rows_v.at[pl.ds(r * 128, 128)], sem).wait()
        pltpu.sync_copy(rows_v, quant_hbm.at[pl.ds(tok0, bpw)])

        base = wid * per_w
        for c in range(nz):
            pltpu.sync_copy(zeros_v, enc_hbm.at[pl.ds(base + c * zlen, zlen)])

        for r in range(nchunks):
            for c in range(8):
                t0 = tok0 + r * 128 + c * 16
                tok = t0 + lax.iota(jnp.int32, 16)
                pos = tok * _NE + idx_v[r, pl.ds(c * 16, 16)]
                pos_v[r, pl.ds(c * 16, 16)] = pos
        for r in range(nchunks):
            pltpu.async_copy(ones_v, enc_hbm.at[pos_v.at[r]], sem).wait()

    return k(weight, idx2)


def kernel(inputs, weight):
    flat_x = jnp.transpose(inputs, (1, 2, 0)).reshape(_NT, _D)
    minval, minidx = _argmin_call(flat_x, weight)
    enc, quant = _sc_encode_gather(weight, minidx)
    dist, loss, ppl = _dist_stats_call(flat_x, weight, minidx, minval)
    out_q = jnp.transpose(quant.reshape(16, 512, _D), (2, 0, 1))
    return (loss.reshape(()),
            out_q,
            ppl.reshape(()),
            enc.reshape(256, 512, 512),
            dist.reshape(256, 512, 512),
            minidx[:, None])


# speedup vs baseline: 2.1342x; 2.1342x over previous
import functools

import jax
import jax.numpy as jnp
from jax import lax
from jax.experimental import pallas as pl
from jax.experimental.pallas import tpu as pltpu
from jax.experimental.pallas import tpu_sc as plsc

_NE = 8192
_D = 256
_NT = 8192
_COMMIT = 0.25

_TB = 128
_A = _TB // 32


def _fused_body(x_ref, w_ref, dist_ref, enc_ref, mi_ref, loss_ref, ppl_ref,
                cnt_s, w2_s, acc_s):
    i = pl.program_id(0)
    ni = pl.num_programs(0)
    x = x_ref[...]
    w = w_ref[...]

    @pl.when(i == 0)
    def _():
        w2_s[...] = jnp.sum(w * w, axis=1)

    x2 = jnp.sum(x * x, axis=1, keepdims=True)
    mm = lax.dot_general(x, w, (((1,), (1,)), ((), ())),
                         preferred_element_type=jnp.float32)
    dist = (x2 + w2_s[...][None, :]) - 2.0 * mm
    dist_ref[...] = dist.reshape(_A, 512, 512)

    tmin = jnp.min(dist, axis=1)
    cols = lax.broadcasted_iota(jnp.int32, (_TB, _NE), 1)
    targ = jnp.min(jnp.where(dist == tmin[:, None], cols, _NE),
                   axis=1)
    mi_ref[...] = targ

    onehot = (targ[:, None] == cols).astype(jnp.float32)
    enc_ref[...] = onehot.reshape(_A, 512, 512)
    colsum = jnp.sum(onehot, axis=0)

    @pl.when(i == 0)
    def _():
        cnt_s[...] = colsum
        acc_s[0] = jnp.sum(tmin)

    @pl.when(i > 0)
    def _():
        cnt_s[...] = cnt_s[...] + colsum
        acc_s[0] = acc_s[0] + jnp.sum(tmin)

    @pl.when(i == ni - 1)
    def _():
        loss_ref[0, 0] = (1.0 + _COMMIT) * acc_s[0] * (1.0 / (_NT * _D))
        p = cnt_s[...] * (1.0 / _NT)
        ppl_ref[0, 0] = jnp.exp(-jnp.sum(p * jnp.log(p + 1e-10)))


def _fused_call(flat_x, weight):
    return pl.pallas_call(
        _fused_body,
        grid=(_NT // _TB,),
        in_specs=[
            pl.BlockSpec((_TB, _D), lambda i: (i, 0)),
            pl.BlockSpec((_NE, _D), lambda i: (0, 0)),
        ],
        out_specs=[
            pl.BlockSpec((_A, 512, 512), lambda i: (i, 0, 0)),
            pl.BlockSpec((_A, 512, 512), lambda i: (i, 0, 0)),
            pl.BlockSpec((_TB,), lambda i: (i,)),
            pl.BlockSpec(memory_space=pltpu.SMEM),
            pl.BlockSpec(memory_space=pltpu.SMEM),
        ],
        out_shape=[
            jax.ShapeDtypeStruct((256, 512, 512), jnp.float32),
            jax.ShapeDtypeStruct((256, 512, 512), jnp.float32),
            jax.ShapeDtypeStruct((_NT,), jnp.int32),
            jax.ShapeDtypeStruct((1, 1), jnp.float32),
            jax.ShapeDtypeStruct((1, 1), jnp.float32),
        ],
        scratch_shapes=[
            pltpu.VMEM((_NE,), jnp.float32),
            pltpu.VMEM((_NE,), jnp.float32),
            pltpu.SMEM((2,), jnp.float32),
        ],
    )(flat_x, weight)


def _sc_gather(weight, minidx):
    info = plsc.get_sparse_core_info()
    nw = info.num_cores * info.num_subcores
    bpw = _NT // nw
    nchunks = bpw // 128
    idx2 = minidx.reshape(_NT // 128, 128)
    mesh = plsc.VectorSubcoreMesh(core_axis_name="c", subcore_axis_name="s")

    @functools.partial(
        pl.kernel, mesh=mesh,
        out_type=jax.ShapeDtypeStruct((_NT, _D), jnp.float32),
        scratch_types=[
            pltpu.VMEM((nchunks, 128), jnp.int32),
            pltpu.VMEM((bpw, _D), jnp.float32),
            pltpu.SemaphoreType.DMA,
        ],
    )
    def k(w_hbm, idx_hbm, out_hbm, idx_v, rows_v, sem):
        wid = lax.axis_index("s") * info.num_cores + lax.axis_index("c")
        pltpu.sync_copy(idx_hbm.at[pl.ds(wid * nchunks, nchunks)], idx_v)
        for r in range(nchunks):
            pltpu.async_copy(w_hbm.at[idx_v.at[r]],
                             rows_v.at[pl.ds(r * 128, 128)], sem).wait()
        pltpu.sync_copy(rows_v, out_hbm.at[pl.ds(wid * bpw, bpw)])

    return k(weight, idx2)


def kernel(inputs, weight):
    flat_x = jnp.transpose(inputs, (1, 2, 0)).reshape(_NT, _D)
    dist3, enc3, minidx, loss, ppl = _fused_call(flat_x, weight)
    quant = _sc_gather(weight, minidx)
    out_q = jnp.transpose(quant.reshape(16, 512, _D), (2, 0, 1))
    return (loss.reshape(()),
            out_q,
            ppl.reshape(()),
            enc3,
            dist3,
            minidx[:, None])


# speedup vs baseline: 2.3994x; 1.1242x over previous
import functools

import jax
import jax.numpy as jnp
from jax import lax
from jax.experimental import pallas as pl
from jax.experimental.pallas import tpu as pltpu
from jax.experimental.pallas import tpu_sc as plsc

_NE = 8192
_D = 256
_NT = 8192
_COMMIT = 0.25

_TB = 256
_A = _TB // 32


def _fused_body(x_ref, w_ref, dist_ref, enc_ref, mi_ref, loss_ref, ppl_ref,
                cnt_s, w2_s, acc_s):
    i = pl.program_id(0)
    ni = pl.num_programs(0)
    x = x_ref[...]
    w = w_ref[...]

    @pl.when(i == 0)
    def _():
        w2_s[...] = jnp.sum(w * w, axis=1)

    x2 = jnp.sum(x * x, axis=1, keepdims=True)
    mm = lax.dot_general(x, w, (((1,), (1,)), ((), ())),
                         preferred_element_type=jnp.float32)
    dist = (x2 + w2_s[...][None, :]) - 2.0 * mm
    dist_ref[...] = dist.reshape(_A, 512, 512)

    tmin = jnp.min(dist, axis=1)
    cols = lax.broadcasted_iota(jnp.int32, (_TB, _NE), 1)
    targ = jnp.min(jnp.where(dist == tmin[:, None], cols, _NE),
                   axis=1)
    mi_ref[...] = targ

    onehot = (targ[:, None] == cols).astype(jnp.float32)
    enc_ref[...] = onehot.reshape(_A, 512, 512)
    colsum = jnp.sum(onehot, axis=0)

    @pl.when(i == 0)
    def _():
        cnt_s[...] = colsum
        acc_s[0] = jnp.sum(tmin)

    @pl.when(i > 0)
    def _():
        cnt_s[...] = cnt_s[...] + colsum
        acc_s[0] = acc_s[0] + jnp.sum(tmin)

    @pl.when(i == ni - 1)
    def _():
        loss_ref[0, 0] = (1.0 + _COMMIT) * acc_s[0] * (1.0 / (_NT * _D))
        p = cnt_s[...] * (1.0 / _NT)
        ppl_ref[0, 0] = jnp.exp(-jnp.sum(p * jnp.log(p + 1e-10)))


def _fused_call(flat_x, weight):
    return pl.pallas_call(
        _fused_body,
        grid=(_NT // _TB,),
        in_specs=[
            pl.BlockSpec((_TB, _D), lambda i: (i, 0)),
            pl.BlockSpec((_NE, _D), lambda i: (0, 0)),
        ],
        out_specs=[
            pl.BlockSpec((_A, 512, 512), lambda i: (i, 0, 0)),
            pl.BlockSpec((_A, 512, 512), lambda i: (i, 0, 0)),
            pl.BlockSpec((_TB,), lambda i: (i,)),
            pl.BlockSpec(memory_space=pltpu.SMEM),
            pl.BlockSpec(memory_space=pltpu.SMEM),
        ],
        out_shape=[
            jax.ShapeDtypeStruct((256, 512, 512), jnp.float32),
            jax.ShapeDtypeStruct((256, 512, 512), jnp.float32),
            jax.ShapeDtypeStruct((_NT,), jnp.int32),
            jax.ShapeDtypeStruct((1, 1), jnp.float32),
            jax.ShapeDtypeStruct((1, 1), jnp.float32),
        ],
        scratch_shapes=[
            pltpu.VMEM((_NE,), jnp.float32),
            pltpu.VMEM((_NE,), jnp.float32),
            pltpu.SMEM((2,), jnp.float32),
        ],
    )(flat_x, weight)


def _sc_gather(weight, minidx):
    info = plsc.get_sparse_core_info()
    nw = info.num_cores * info.num_subcores
    bpw = _NT // nw
    nchunks = bpw // 128
    idx2 = minidx.reshape(_NT // 128, 128)
    mesh = plsc.VectorSubcoreMesh(core_axis_name="c", subcore_axis_name="s")

    @functools.partial(
        pl.kernel, mesh=mesh,
        out_type=jax.ShapeDtypeStruct((_NT, _D), jnp.float32),
        scratch_types=[
            pltpu.VMEM((nchunks, 128), jnp.int32),
            pltpu.VMEM((bpw, _D), jnp.float32),
            pltpu.SemaphoreType.DMA,
        ],
    )
    def k(w_hbm, idx_hbm, out_hbm, idx_v, rows_v, sem):
        wid = lax.axis_index("s") * info.num_cores + lax.axis_index("c")
        pltpu.sync_copy(idx_hbm.at[pl.ds(wid * nchunks, nchunks)], idx_v)
        for r in range(nchunks):
            pltpu.async_copy(w_hbm.at[idx_v.at[r]],
                             rows_v.at[pl.ds(r * 128, 128)], sem).wait()
        pltpu.sync_copy(rows_v, out_hbm.at[pl.ds(wid * bpw, bpw)])

    return k(weight, idx2)


def kernel(inputs, weight):
    flat_x = jnp.transpose(inputs, (1, 2, 0)).reshape(_NT, _D)
    dist3, enc3, minidx, loss, ppl = _fused_call(flat_x, weight)
    quant = _sc_gather(weight, minidx)
    out_q = jnp.transpose(quant.reshape(16, 512, _D), (2, 0, 1))
    return (loss.reshape(()),
            out_q,
            ppl.reshape(()),
            enc3,
            dist3,
            minidx[:, None])
